# baseline (speedup 1.0000x reference)
import functools

import jax
import jax.numpy as jnp
from jax import lax
from jax.experimental import pallas as pl
from jax.experimental.pallas import tpu as pltpu
from jax.experimental.pallas import tpu_sc as plsc

_E = 64
_C = 2
_NC = 2
_NS = 16
_NW = _NC * _NS
_G = 128
_L = 16


def _make_kernel(n_tokens):
    tok_per_w = n_tokens // _NW
    n_groups = tok_per_w // _G
    mesh = plsc.VectorSubcoreMesh(core_axis_name="c", subcore_axis_name="s")

    @functools.partial(
        pl.kernel,
        out_type=jax.ShapeDtypeStruct((_C, _NW, tok_per_w), jnp.float32),
        mesh=mesh,
        compiler_params=pltpu.CompilerParams(
            needs_layout_passes=False, use_tc_tiling_on_sc=False),
        scratch_types=[
            pltpu.VMEM((n_groups, _G), jnp.int32),
            pltpu.VMEM((_G, _E), jnp.float32),
            pltpu.VMEM((tok_per_w,), jnp.float32),
            pltpu.VMEM((tok_per_w,), jnp.float32),
            pltpu.VMEM((_C, _E), jnp.float32),
            pltpu.VMEM((16,), jnp.float32),
            pltpu.SemaphoreType.DMA,
        ],
    )
    def k(idx_hbm, table_hbm, w_hbm, b_hbm, out_hbm,
          idx_v, rows_v, out0_v, out1_v, w_v, b_v, sem):
        wid = lax.axis_index("s") * _NC + lax.axis_index("c")
        pltpu.sync_copy(idx_hbm.at[wid], idx_v)
        pltpu.sync_copy(w_hbm, w_v)
        pltpu.sync_copy(b_hbm, b_v.at[pl.ds(0, _C)])

        bvec = b_v[...]
        lanes = lax.iota(jnp.int32, _L)
        lane0 = lanes == 0
        lane_last = lanes == (_L - 1)
        b0v = jnp.where(lane0, bvec[0], 0.0)
        b1v = jnp.where(lane0, bvec[1], 0.0)
        w0 = [w_v[0, pl.ds(_L * q, _L)] for q in range(_E // _L)]
        w1 = [w_v[1, pl.ds(_L * q, _L)] for q in range(_E // _L)]

        def group_body(j, _):
            pltpu.async_copy(table_hbm.at[idx_v.at[j]], rows_v, sem).wait()

            def tok_body(t, _):
                r = [rows_v[t, pl.ds(_L * q, _L)] for q in range(_E // _L)]
                acc0 = b0v + r[0] * w0[0]
                acc1 = b1v + r[0] * w1[0]
                for q in range(1, _E // _L):
                    acc0 = acc0 + r[q] * w0[q]
                    acc1 = acc1 + r[q] * w1[q]
                pos = jnp.full((_L,), j * _G + t, jnp.int32)
                plsc.store_scatter(out0_v, [pos], plsc.cumsum(acc0),
                                   mask=lane_last)
                plsc.store_scatter(out1_v, [pos], plsc.cumsum(acc1),
                                   mask=lane_last)
                return 0

            lax.fori_loop(0, _G, tok_body, 0)
            return 0

        lax.fori_loop(0, n_groups, group_body, 0)
        pltpu.sync_copy(out0_v, out_hbm.at[0, wid])
        pltpu.sync_copy(out1_v, out_hbm.at[1, wid])

    return k


@jax.jit
def kernel(sentence1, emb_table, W, b):
    batch, seq = sentence1.shape
    n_tokens = batch * seq
    tok_per_w = n_tokens // _NW
    idx = sentence1.reshape(_NW, tok_per_w // _G, _G)
    out = _make_kernel(n_tokens)(idx, emb_table, W, b)
    return out.reshape(_C, n_tokens).T.reshape(batch, seq, _C)


# speedup vs baseline: 1.2184x; 1.2184x over previous
import functools

import jax
import jax.numpy as jnp
from jax import lax
from jax.experimental import pallas as pl
from jax.experimental.pallas import tpu as pltpu
from jax.experimental.pallas import tpu_sc as plsc

_E = 64
_C = 2
_NC = 2
_NS = 16
_NW = _NC * _NS
_G = 128
_BLK = 6400


def _project_kernel(x_ref, w_ref, b_ref, out_ref):
    res = lax.dot_general(
        w_ref[...], x_ref[...], (((1,), (1,)), ((), ())),
        preferred_element_type=jnp.float32)
    out_ref[...] = res + b_ref[...]


def _project(table, Wp, bp):
    vocab = table.shape[0]
    grid = (vocab + _BLK - 1) // _BLK
    return pl.pallas_call(
        _project_kernel,
        grid=(grid,),
        in_specs=[
            pl.BlockSpec((_BLK, _E), lambda i: (i, 0)),
            pl.BlockSpec((8, _E), lambda i: (0, 0)),
            pl.BlockSpec((8, 1), lambda i: (0, 0)),
        ],
        out_specs=pl.BlockSpec((8, _BLK), lambda i: (0, i)),
        out_shape=jax.ShapeDtypeStruct((8, vocab), jnp.float32),
    )(table, Wp, bp)


def _make_gather(n_tokens):
    tok_per_w = n_tokens // _NW
    n_groups = tok_per_w // _G
    mesh = plsc.VectorSubcoreMesh(core_axis_name="c", subcore_axis_name="s")

    @functools.partial(
        pl.kernel,
        out_type=jax.ShapeDtypeStruct((_C, _NW, tok_per_w), jnp.float32),
        mesh=mesh,
        compiler_params=pltpu.CompilerParams(
            needs_layout_passes=False, use_tc_tiling_on_sc=False),
        scratch_types=[
            pltpu.VMEM((tok_per_w,), jnp.int32),
            pltpu.VMEM((tok_per_w,), jnp.float32),
            pltpu.VMEM((tok_per_w,), jnp.float32),
            pltpu.SemaphoreType.DMA,
        ],
    )
    def k(p0_hbm, p1_hbm, idx_hbm, out_hbm, idx_v, s0_v, s1_v, sem):
        wid = lax.axis_index("s") * _NC + lax.axis_index("c")
        pltpu.sync_copy(idx_hbm.at[wid], idx_v)
        handles = []
        for j in range(n_groups):
            sl = pl.ds(j * _G, _G)
            handles.append(
                pltpu.async_copy(p0_hbm.at[idx_v.at[sl]], s0_v.at[sl], sem))
            handles.append(
                pltpu.async_copy(p1_hbm.at[idx_v.at[sl]], s1_v.at[sl], sem))
        for h in handles:
            h.wait()
        pltpu.sync_copy(s0_v, out_hbm.at[0, wid])
        pltpu.sync_copy(s1_v, out_hbm.at[1, wid])

    return k


@jax.jit
def kernel(sentence1, emb_table, W, b):
    batch, seq = sentence1.shape
    n_tokens = batch * seq
    Wp = jnp.zeros((8, _E), jnp.float32).at[:_C].set(W)
    bp = jnp.zeros((8, 1), jnp.float32).at[:_C, 0].set(b)
    proj = _project(emb_table, Wp, bp)
    p0 = proj[0]
    p1 = proj[1]
    idx = sentence1.reshape(_NW, n_tokens // _NW)
    out = _make_gather(n_tokens)(p0, p1, idx)
    return out.reshape(_C, n_tokens).T.reshape(batch, seq, _C)
